# baseline (speedup 1.0000x reference)
import functools

import jax
import jax.numpy as jnp
from jax import lax
from jax.experimental import pallas as pl
from jax.experimental.pallas import tpu as pltpu
from jax.experimental.pallas import tpu_sc as plsc

B, C, H, W = 8, 96, 224, 224
R = B * C
D = H * W
NC, NS = 2, 16
NW = NC * NS
RPW = R // NW


def _body(x_hbm, idx_hbm, out_hbm, idx_v, buf0, buf1, gsem0, gsem1,
          ssem0, ssem1):
    wid = lax.axis_index("s") * NC + lax.axis_index("c")
    base = wid * RPW
    pltpu.sync_copy(idx_hbm.at[pl.ds(base, RPW)], idx_v)

    bufs = (buf0, buf1)
    gsems = (gsem0, gsem1)
    ssems = (ssem0, ssem1)

    def gather(j):
        b = j % 2
        return pltpu.async_copy(x_hbm.at[idx_v.at[j]], bufs[b], gsems[b])

    def scatter(j):
        b = j % 2
        return pltpu.async_copy(bufs[b], out_hbm.at[pl.ds(base + j, 1)],
                                ssems[b])

    gather(0)
    gather(1)
    for j in range(RPW):
        b = j % 2
        pltpu.make_async_copy(x_hbm.at[idx_v.at[j]], bufs[b],
                              gsems[b]).wait()
        sc = scatter(j)
        if j + 2 < RPW:
            sc.wait()
            gather(j + 2)
    for j in (RPW - 2, RPW - 1):
        b = j % 2
        pltpu.make_async_copy(bufs[b], out_hbm.at[pl.ds(base + j, 1)],
                              ssems[b]).wait()


@jax.jit
def kernel(x, perm):
    x2 = x.reshape(R, D)
    rows = jnp.arange(R, dtype=jnp.int32)
    src = (rows // C) * C + perm.astype(jnp.int32)[rows % C]
    src = src.reshape(R, 1)

    mesh = plsc.VectorSubcoreMesh(core_axis_name="c", subcore_axis_name="s")
    out2 = pl.kernel(
        _body,
        out_type=jax.ShapeDtypeStruct((R, D), jnp.float32),
        mesh=mesh,
        scratch_types=[
            pltpu.VMEM((RPW, 1), jnp.int32),
            pltpu.VMEM((1, D), jnp.float32),
            pltpu.VMEM((1, D), jnp.float32),
            pltpu.SemaphoreType.DMA,
            pltpu.SemaphoreType.DMA,
            pltpu.SemaphoreType.DMA,
            pltpu.SemaphoreType.DMA,
        ],
    )(x2, src)
    return out2.reshape(B, C, H, W)
